# baseline (speedup 1.0000x reference)
import functools

import jax
import jax.numpy as jnp
from jax import lax
from jax.experimental import pallas as pl
from jax.experimental.pallas import tpu as pltpu
from jax.experimental.pallas import tpu_sc as plsc

B, C, H, W = 32, 96, 112, 112
NC, NS, VEC = 2, 16, 16
NWORK = NC * NS
PCH = C // NWORK
WV = W // VEC
ITERS = PCH * H * WV


def _sc_body(x_ref, inter_ref, out_ref, inter_v, buf0, buf1,
             sem_in0, sem_in1, sem_out0, sem_out1):
    wid = lax.axis_index("s") * NC + lax.axis_index("c")
    c0 = wid * PCH
    sl = pl.ds(c0, PCH)
    bufs = (buf0, buf1)
    sem_in = (sem_in0, sem_in1)
    sem_out = (sem_out0, sem_out1)
    pltpu.sync_copy(inter_ref.at[sl], inter_v)

    def compute(buf):
        @plsc.parallel_loop(0, ITERS, step=1, unroll=8)
        def _(i):
            c = i // (H * WV)
            r = i - c * (H * WV)
            h = r // WV
            w0 = (r - h * WV) * VEC
            xv = buf[c, h, pl.ds(w0, VEC)]
            iv = inter_v[c, h, pl.ds(w0, VEC)]
            x_inter = xv * (1.0 - iv) + xv * iv
            buf[c, h, pl.ds(w0, VEC)] = jnp.where(x_inter > 0.0, xv, 0.0)

    in_h = [None, None]
    out_h = [None, None]
    in_h[0] = pltpu.async_copy(x_ref.at[0, sl], bufs[0], sem_in[0])
    for b in range(B):
        k = b % 2
        nk = (b + 1) % 2
        in_h[k].wait()
        if b + 1 < B:
            if out_h[nk] is not None:
                out_h[nk].wait()
            in_h[nk] = pltpu.async_copy(x_ref.at[b + 1, sl], bufs[nk], sem_in[nk])
        compute(bufs[k])
        out_h[k] = pltpu.async_copy(bufs[k], out_ref.at[b, sl], sem_out[k])
    out_h[0].wait()
    out_h[1].wait()


def kernel(x, prototype, inter):
    del prototype
    run = functools.partial(
        pl.kernel,
        mesh=plsc.VectorSubcoreMesh(core_axis_name="c", subcore_axis_name="s"),
        out_type=jax.ShapeDtypeStruct((B, C, H, W), jnp.float32),
        scratch_types=[
            pltpu.VMEM((PCH, H, W), jnp.float32),
            pltpu.VMEM((PCH, H, W), jnp.float32),
            pltpu.VMEM((PCH, H, W), jnp.float32),
            pltpu.SemaphoreType.DMA,
            pltpu.SemaphoreType.DMA,
            pltpu.SemaphoreType.DMA,
            pltpu.SemaphoreType.DMA,
        ],
        compiler_params=pltpu.CompilerParams(use_tc_tiling_on_sc=True),
    )(_sc_body)
    return run(x, inter)


# speedup vs baseline: 3.3814x; 3.3814x over previous
import functools

import jax
import jax.numpy as jnp
from jax import lax
from jax.experimental import pallas as pl
from jax.experimental.pallas import tpu as pltpu
from jax.experimental.pallas import tpu_sc as plsc

B, C, H, W = 32, 96, 112, 112
NC, NS, VEC = 2, 16, 16
NWORK = NC * NS
PCH = C // NWORK
WV = W // VEC
ITERS = PCH * H * WV


def _sc_body(x_ref, inter_ref, out_ref, inter_v, buf0, buf1,
             sem_in0, sem_in1, sem_out0, sem_out1):
    wid = lax.axis_index("s") * NC + lax.axis_index("c")
    c0 = wid * PCH
    sl = pl.ds(c0, PCH)
    bufs = (buf0, buf1)
    sem_in = (sem_in0, sem_in1)
    sem_out = (sem_out0, sem_out1)
    pltpu.sync_copy(inter_ref.at[sl], inter_v)

    def compute(buf):
        @plsc.parallel_loop(0, PCH * H, step=1, unroll=1)
        def _(i):
            c = i // H
            h = i - c * H
            for v in range(WV):
                w0 = v * VEC
                xv = buf[c, h, pl.ds(w0, VEC)]
                iv = inter_v[c, h, pl.ds(w0, VEC)]
                x_inter = xv * (1.0 - iv) + xv * iv
                buf[c, h, pl.ds(w0, VEC)] = jnp.where(x_inter > 0.0, xv, 0.0)

    in_h = [None, None]
    out_h = [None, None]
    in_h[0] = pltpu.async_copy(x_ref.at[0, sl], bufs[0], sem_in[0])
    for b in range(B):
        k = b % 2
        nk = (b + 1) % 2
        in_h[k].wait()
        if b + 1 < B:
            if out_h[nk] is not None:
                out_h[nk].wait()
            in_h[nk] = pltpu.async_copy(x_ref.at[b + 1, sl], bufs[nk], sem_in[nk])
        compute(bufs[k])
        out_h[k] = pltpu.async_copy(bufs[k], out_ref.at[b, sl], sem_out[k])
    out_h[0].wait()
    out_h[1].wait()


def kernel(x, prototype, inter):
    del prototype
    run = functools.partial(
        pl.kernel,
        mesh=plsc.VectorSubcoreMesh(core_axis_name="c", subcore_axis_name="s"),
        out_type=jax.ShapeDtypeStruct((B, C, H, W), jnp.float32),
        scratch_types=[
            pltpu.VMEM((PCH, H, W), jnp.float32),
            pltpu.VMEM((PCH, H, W), jnp.float32),
            pltpu.VMEM((PCH, H, W), jnp.float32),
            pltpu.SemaphoreType.DMA,
            pltpu.SemaphoreType.DMA,
            pltpu.SemaphoreType.DMA,
            pltpu.SemaphoreType.DMA,
        ],
        compiler_params=pltpu.CompilerParams(use_tc_tiling_on_sc=True),
    )(_sc_body)
    return run(x, inter)
